# baseline (speedup 1.0000x reference)
import functools
import jax
import jax.numpy as jnp
from jax import lax
from jax.experimental import pallas as pl
from jax.experimental.pallas import tpu as pltpu
from jax.experimental.pallas import tpu_sc as plsc

_VOCAB = 1000000
_EMB = 64
_OUT = 2
_L = 200
_B = 4096

_NC = 2
_NS = 16
_NW = _NC * _NS
_NB = _B // _NW
_LN = 16
_C0, _C1 = 96, 104


def _make_sc_kernel():
    mesh = plsc.VectorSubcoreMesh(core_axis_name="c", subcore_axis_name="s")

    @functools.partial(
        pl.kernel,
        mesh=mesh,
        compiler_params=pltpu.CompilerParams(use_tc_tiling_on_sc=False),
        out_type=jax.ShapeDtypeStruct((_B * _EMB,), jnp.float32),
        scratch_types=[
            pltpu.VMEM((_NB * _L,), jnp.int32),
            pltpu.VMEM((_L, _EMB), jnp.float32),
            pltpu.VMEM((_NB * _EMB,), jnp.float32),
            pltpu.SemaphoreType.DMA,
        ],
    )
    def sc_sum_kernel(xTf_hbm, table_hbm, out_hbm,
                      idx_flat, rows_v, sums_flat, sem):
        wid = lax.axis_index("s") * _NC + lax.axis_index("c")
        base = wid * _NB
        ioff = pl.multiple_of(base * _L, 8)
        pltpu.sync_copy(xTf_hbm.at[pl.ds(ioff, _NB * _L)], idx_flat)

        zero = jnp.zeros((_LN,), jnp.float32)

        def batch_body(j, carry):
            joff = pl.multiple_of(j * _L, 8)
            cp0 = pltpu.async_copy(
                table_hbm.at[idx_flat.at[pl.ds(joff, _C0)]],
                rows_v.at[pl.ds(0, _C0)], sem)
            joff2 = pl.multiple_of(joff + _C0, 8)
            cp1 = pltpu.async_copy(
                table_hbm.at[idx_flat.at[pl.ds(joff2, _C1)]],
                rows_v.at[pl.ds(_C0, _C1)], sem)
            cp0.wait()
            cp1.wait()

            def red(l, acc):
                return tuple(
                    acc[d] + rows_v[l, pl.ds(d * _LN, _LN)]
                    for d in range(4)
                )

            acc = lax.fori_loop(0, _L, red, (zero, zero, zero, zero))
            soff = pl.multiple_of(j * _EMB, 8)
            for d in range(4):
                sums_flat[pl.ds(soff + d * _LN, _LN)] = acc[d]
            return carry

        lax.fori_loop(0, _NB, batch_body, 0)

        ooff = pl.multiple_of(base * _EMB, 8)
        pltpu.sync_copy(sums_flat, out_hbm.at[pl.ds(ooff, _NB * _EMB)])

    return sc_sum_kernel


_SC_SUM = _make_sc_kernel()


def _tc_proj_body(sums_ref, w_ref, b_ref, out_ref):
    p = sums_ref[...] * jnp.float32(1.0 / _L)
    out = lax.dot_general(p, w_ref[...], (((1,), (1,)), ((), ())),
                          preferred_element_type=jnp.float32)
    out_ref[...] = out + b_ref[...]


_TC_PROJ = pl.pallas_call(
    _tc_proj_body,
    out_shape=jax.ShapeDtypeStruct((_B, _OUT), jnp.float32),
)


@jax.jit
def kernel(x, table, W, b):
    xTf = jnp.asarray(x, jnp.int32).T.reshape(-1)
    sums = _SC_SUM(xTf, table).reshape(_B, _EMB)
    return _TC_PROJ(sums, W.astype(jnp.float32),
                    b.astype(jnp.float32)[None, :])


# speedup vs baseline: 1.1693x; 1.1693x over previous
import functools
import jax
import jax.numpy as jnp
from jax import lax
from jax.experimental import pallas as pl
from jax.experimental.pallas import tpu as pltpu
from jax.experimental.pallas import tpu_sc as plsc

_VOCAB = 1000000
_EMB = 64
_OUT = 2
_L = 200
_B = 4096

_NC = 2
_NS = 16
_NW = _NC * _NS
_NB = _B // _NW
_LN = 16
_C0, _C1 = 96, 104


def _make_sc_kernel():
    mesh = plsc.VectorSubcoreMesh(core_axis_name="c", subcore_axis_name="s")

    @functools.partial(
        pl.kernel,
        mesh=mesh,
        compiler_params=pltpu.CompilerParams(use_tc_tiling_on_sc=False),
        out_type=jax.ShapeDtypeStruct((_B * _EMB,), jnp.float32),
        scratch_types=[
            pltpu.VMEM((_NB * _L,), jnp.int32),
            pltpu.VMEM((_L, _EMB), jnp.float32),
            pltpu.VMEM((_L, _EMB), jnp.float32),
            pltpu.VMEM((_NB * _EMB,), jnp.float32),
            pltpu.SemaphoreType.DMA,
            pltpu.SemaphoreType.DMA,
        ],
    )
    def sc_sum_kernel(xTf_hbm, table_hbm, out_hbm,
                      idx_flat, buf0, buf1, sums_flat, sem0, sem1):
        wid = lax.axis_index("s") * _NC + lax.axis_index("c")
        base = wid * _NB
        ioff = pl.multiple_of(base * _L, 8)
        pltpu.sync_copy(xTf_hbm.at[pl.ds(ioff, _NB * _L)], idx_flat)

        zero = jnp.zeros((_LN,), jnp.float32)

        def issue(j, buf, sem):
            joff = pl.multiple_of(j * _L, 8)
            pltpu.async_copy(table_hbm.at[idx_flat.at[pl.ds(joff, _C0)]],
                             buf.at[pl.ds(0, _C0)], sem)
            joff2 = pl.multiple_of(joff + _C0, 8)
            pltpu.async_copy(table_hbm.at[idx_flat.at[pl.ds(joff2, _C1)]],
                             buf.at[pl.ds(_C0, _C1)], sem)

        def wait(buf, sem):
            pltpu.make_async_copy(table_hbm.at[pl.ds(0, _L)], buf, sem).wait()

        def reduce_store(buf, j):
            def red(l, acc):
                return tuple(
                    acc[d] + buf[l, pl.ds(d * _LN, _LN)] for d in range(4)
                )

            acc = lax.fori_loop(0, _L, red, (zero, zero, zero, zero),
                                unroll=8)
            soff = pl.multiple_of(j * _EMB, 8)
            for d in range(4):
                sums_flat[pl.ds(soff + d * _LN, _LN)] = acc[d]

        issue(0, buf0, sem0)
        issue(1, buf1, sem1)

        def body(jj, carry):
            j = 2 * jj
            wait(buf0, sem0)
            reduce_store(buf0, j)
            issue(j + 2, buf0, sem0)
            wait(buf1, sem1)
            reduce_store(buf1, j + 1)
            issue(j + 3, buf1, sem1)
            return carry

        lax.fori_loop(0, _NB // 2 - 1, body, 0)

        wait(buf0, sem0)
        reduce_store(buf0, _NB - 2)
        wait(buf1, sem1)
        reduce_store(buf1, _NB - 1)

        ooff = pl.multiple_of(base * _EMB, 8)
        pltpu.sync_copy(sums_flat, out_hbm.at[pl.ds(ooff, _NB * _EMB)])

    return sc_sum_kernel


_SC_SUM = _make_sc_kernel()


def _tc_proj_body(sums_ref, w_ref, b_ref, out_ref):
    p = sums_ref[...] * jnp.float32(1.0 / _L)
    out = lax.dot_general(p, w_ref[...], (((1,), (1,)), ((), ())),
                          preferred_element_type=jnp.float32)
    out_ref[...] = out + b_ref[...]


_TC_PROJ = pl.pallas_call(
    _tc_proj_body,
    out_shape=jax.ShapeDtypeStruct((_B, _OUT), jnp.float32),
)


@jax.jit
def kernel(x, table, W, b):
    xTf = jnp.asarray(x, jnp.int32).T.reshape(-1)
    sums = _SC_SUM(xTf, table).reshape(_B, _EMB)
    return _TC_PROJ(sums, W.astype(jnp.float32),
                    b.astype(jnp.float32)[None, :])
